# baseline (speedup 1.0000x reference)
import jax
import jax.numpy as jnp
from jax import lax
from jax.experimental import pallas as pl
from jax.experimental.pallas import tpu as pltpu
from jax.experimental.pallas import tpu_sc as plsc

NC = 2
NS = 16
NW = NC * NS
L = 16
CHUNK = 96


def _matmul_body(x_ref, wd_ref, w2_ref, y1_ref, y2_ref):
    x = x_ref[...]
    dn = (((0,), (0,)), ((), ()))
    y1_ref[...] = lax.dot_general(x, wd_ref[...], dn,
                                  preferred_element_type=jnp.float32)
    y2_ref[...] = lax.dot_general(x, w2_ref[...], dn,
                                  preferred_element_type=jnp.float32)


def _stage1(X, Wd_t, W2_t):
    C, N = X.shape
    O = Wd_t.shape[1]
    return pl.pallas_call(
        _matmul_body,
        out_shape=[
            jax.ShapeDtypeStruct((N, O), jnp.float32),
            jax.ShapeDtypeStruct((N, O), jnp.float32),
        ],
    )(X, Wd_t, W2_t)


def _make_sc_kernel(N, O, NPAD, CPW):
    assert CPW % 3 == 0
    blocks = CPW // 3
    row_grps = N // CHUNK
    zero_tail = NPAD - row_grps * CHUNK
    pub_tail = N - row_grps * CHUNK
    assert zero_tail % 8 == 0 and pub_tail % 8 == 0

    mesh = plsc.VectorSubcoreMesh(core_axis_name="c", subcore_axis_name="s")

    def body(y2_hbm, ridx_hbm, gidx_hbm, z_hbm, zhist_hbm,
             s_out, cnt_out,
             s_sh, idxg0, idxg1, idxg2, idxr0, idxr1, idxr2,
             scx0, scx1, rows0, rows1, rows2, hist_v,
             semg0, semg1, semg2, semi0, semi1, semi2, sems0, sems1):
        cid = lax.axis_index("c")
        sid = lax.axis_index("s")
        wid = sid * NC + cid
        c0 = wid * CPW

        idxg = (idxg0, idxg1, idxg2)
        idxr = (idxr0, idxr1, idxr2)
        scx = (scx0, scx1)
        rows = (rows0, rows1, rows2)
        semg = (semg0, semg1, semg2)
        semi = (semi0, semi1, semi2)
        sems = (sems0, sems1)

        pltpu.sync_copy(z_hbm, rows0)
        pltpu.sync_copy(zhist_hbm, hist_v)

        def zero_grp(g, carry):
            o = (sid + g * NS) * CHUNK
            pltpu.sync_copy(rows0, s_sh.at[pl.ds(o, CHUNK)])
            return carry

        lax.fori_loop(0, row_grps // NS, zero_grp, 0)
        rem = row_grps % NS

        @pl.when(sid < rem)
        def _zero_rem():
            o = ((row_grps // NS) * NS + sid) * CHUNK
            pltpu.sync_copy(rows0, s_sh.at[pl.ds(o, CHUNK)])

        if zero_tail:
            @pl.when(sid == NS - 1)
            def _zero_tail():
                t0 = row_grps * CHUNK
                pltpu.sync_copy(rows0.at[pl.ds(0, zero_tail)],
                                s_sh.at[pl.ds(t0, zero_tail)])
        plsc.subcore_barrier()

        def prefetch_idx(j, s):
            base = (c0 + j) * CHUNK
            pltpu.make_async_copy(gidx_hbm.at[pl.ds(base, CHUNK)],
                                  idxg[s], semi[s]).start()
            pltpu.make_async_copy(ridx_hbm.at[pl.ds(base, CHUNK)],
                                  idxr[s], semi[s]).start()

        def wait_idx(s):
            pltpu.make_async_copy(gidx_hbm.at[pl.ds(0, CHUNK)],
                                  idxg[s], semi[s]).wait()
            pltpu.make_async_copy(ridx_hbm.at[pl.ds(0, CHUNK)],
                                  idxr[s], semi[s]).wait()

        def wait_scatter(u, v):
            pltpu.make_async_copy(rows[u], s_sh.at[scx[v]], sems[v]).wait()

        def steady(jb, u, v, first=False):
            un, up = (u + 1) % 3, (u + 2) % 3
            wait_idx(un)
            if not first:
                wait_scatter(un, v)
            pltpu.make_async_copy(y2_hbm.at[idxg[un]], rows[un],
                                  semg[un]).start()
            prefetch_idx(jb + u + 2, up)
            for q in range(CHUNK // L):
                iv = idxr[u][pl.ds(q * L, L)]
                scx[v][pl.ds(q * L, L)] = iv
                cnts, last = plsc.scan_count(iv)
                plsc.addupdate_scatter(hist_v, [iv],
                                       cnts.astype(jnp.float32), mask=last)
            pltpu.make_async_copy(y2_hbm.at[idxg[u]], rows[u],
                                  semg[u]).wait()
            pltpu.async_copy(rows[u], s_sh.at[scx[v]], sems[v],
                             add=True)

        prefetch_idx(0, 0)
        wait_idx(0)
        prefetch_idx(1, 1)
        pltpu.make_async_copy(y2_hbm.at[idxg0], rows0, semg0).start()

        steady(0, 0, 0, first=True)
        steady(0, 1, 1, first=True)
        steady(0, 2, 0)

        def block(t, carry):
            jb = t * 3
            steady(jb, 0, 1)
            steady(jb, 1, 0)
            steady(jb, 2, 1)
            return carry

        def block_pair(t, carry):
            jb = t * 6 + 3
            steady(jb, 0, 1)
            steady(jb, 1, 0)
            steady(jb, 2, 1)
            steady(jb + 3, 0, 0)
            steady(jb + 3, 1, 1)
            steady(jb + 3, 2, 0)
            return carry

        del block
        lax.fori_loop(0, (blocks - 1) // 2, block_pair, 0)
        last = CPW - 1
        pltpu.make_async_copy(y2_hbm.at[idxg[(last + 1) % 3]],
                              rows[(last + 1) % 3],
                              semg[(last + 1) % 3]).wait()
        wait_idx((last + 2) % 3)
        wait_scatter((last - 1) % 3, (last - 1) % 2)
        wait_scatter(last % 3, last % 2)
        plsc.subcore_barrier()

        def pub_grp(g, carry):
            o = (sid + g * NS) * CHUNK
            pltpu.sync_copy(s_sh.at[pl.ds(o, CHUNK)], rows0)
            pltpu.sync_copy(rows0, s_out.at[cid, pl.ds(o, CHUNK)])
            return carry

        lax.fori_loop(0, row_grps // NS, pub_grp, 0)

        @pl.when(sid < rem)
        def _pub_rem():
            o = ((row_grps // NS) * NS + sid) * CHUNK
            pltpu.sync_copy(s_sh.at[pl.ds(o, CHUNK)], rows0)
            pltpu.sync_copy(rows0, s_out.at[cid, pl.ds(o, CHUNK)])

        if pub_tail:
            @pl.when(sid == NS - 1)
            def _pub_tail():
                t0 = row_grps * CHUNK
                pltpu.sync_copy(s_sh.at[pl.ds(t0, pub_tail)],
                                rows0.at[pl.ds(0, pub_tail)])
                pltpu.sync_copy(rows0.at[pl.ds(0, pub_tail)],
                                s_out.at[cid, pl.ds(t0, pub_tail)])

        pltpu.sync_copy(hist_v, cnt_out.at[cid, sid])

    return pl.kernel(
        body,
        out_type=[
            jax.ShapeDtypeStruct((NC, N, O), jnp.float32),
            jax.ShapeDtypeStruct((NC, NS, NPAD), jnp.float32),
        ],
        mesh=mesh,
        compiler_params=pltpu.CompilerParams(needs_layout_passes=False),
        scratch_types=[
            pltpu.VMEM_SHARED((NPAD, O), jnp.float32),
            pltpu.VMEM((CHUNK,), jnp.int32),
            pltpu.VMEM((CHUNK,), jnp.int32),
            pltpu.VMEM((CHUNK,), jnp.int32),
            pltpu.VMEM((CHUNK,), jnp.int32),
            pltpu.VMEM((CHUNK,), jnp.int32),
            pltpu.VMEM((CHUNK,), jnp.int32),
            pltpu.VMEM((CHUNK,), jnp.int32),
            pltpu.VMEM((CHUNK,), jnp.int32),
            pltpu.VMEM((CHUNK, O), jnp.float32),
            pltpu.VMEM((CHUNK, O), jnp.float32),
            pltpu.VMEM((CHUNK, O), jnp.float32),
            pltpu.VMEM((NPAD,), jnp.float32),
            pltpu.SemaphoreType.DMA,
            pltpu.SemaphoreType.DMA,
            pltpu.SemaphoreType.DMA,
            pltpu.SemaphoreType.DMA,
            pltpu.SemaphoreType.DMA,
            pltpu.SemaphoreType.DMA,
            pltpu.SemaphoreType.DMA,
            pltpu.SemaphoreType.DMA,
        ],
    )


def _combine_body(y1_ref, s_ref, cnt_ref, b_ref, pw_ref, out_ref):
    s = s_ref[0] + s_ref[1]
    c = jnp.sum(cnt_ref[...], axis=0)[:, None]
    y = y1_ref[...] + b_ref[...]
    tot = c * y + s
    out = tot / jnp.maximum(c, 1.0)
    pw = pw_ref[0, 0]
    out = jnp.where(out >= 0, out, pw * out)
    out_ref[...] = out.T


def _stage3(Y1t, S, CNT, b2, pw2):
    N, O = Y1t.shape
    return pl.pallas_call(
        _combine_body,
        out_shape=jax.ShapeDtypeStruct((O, N), jnp.float32),
    )(Y1t, S, CNT, b2, pw2)


def kernel(in_features, reduce_index, gather_index, W, b, prelu_w):
    X = in_features[0]
    C, N = X.shape
    O = W.shape[0]
    E = reduce_index.shape[0]
    NPAD = N + 8
    CPW = -(-E // (NW * CHUNK))
    while CPW % 6 != 3:
        CPW += 1
    EPAD = CPW * NW * CHUNK
    EALLOC = EPAD + 2 * CHUNK

    ridx = reduce_index.astype(jnp.int32)
    gidx = gather_index.astype(jnp.int32)
    rpad = jnp.concatenate([ridx, jnp.full((EALLOC - E,), N, jnp.int32)])
    gpad = jnp.concatenate([gidx, jnp.zeros((EALLOC - E,), jnp.int32)])

    W1 = W[:, :C]
    W2 = W[:, C:]
    Wd_t = (W1 - W2).T
    W2_t = W2.T

    Y1t, Y2t = _stage1(X, Wd_t, W2_t)

    z = jnp.zeros((CHUNK, O), jnp.float32)
    zhist = jnp.zeros((NPAD,), jnp.float32)
    S, CNT = _make_sc_kernel(N, O, NPAD, CPW)(Y2t, rpad, gpad, z, zhist)

    out2d = _stage3(Y1t, S, CNT[:, :, :N].reshape(NC * NS, N),
                    b.reshape(1, O), prelu_w.reshape(1, 1))
    return out2d[None]


# speedup vs baseline: 1.5805x; 1.5805x over previous
import jax
import jax.numpy as jnp
from jax import lax
from jax.experimental import pallas as pl
from jax.experimental.pallas import tpu as pltpu
from jax.experimental.pallas import tpu_sc as plsc

NC = 2
NS = 16
NW = NC * NS
L = 16
CHUNK = 128


def _matmul_body(x_ref, wd_ref, w2_ref, y1_ref, y2_ref):
    x = x_ref[...]
    dn = (((0,), (0,)), ((), ()))
    y1_ref[...] = lax.dot_general(x, wd_ref[...], dn,
                                  preferred_element_type=jnp.float32)
    y2_ref[...] = lax.dot_general(x, w2_ref[...], dn,
                                  preferred_element_type=jnp.float32)


def _stage1(X, Wd_t, W2_t):
    C, N = X.shape
    O = Wd_t.shape[1]
    return pl.pallas_call(
        _matmul_body,
        out_shape=[
            jax.ShapeDtypeStruct((N, O), jnp.float32),
            jax.ShapeDtypeStruct((N, O), jnp.float32),
        ],
    )(X, Wd_t, W2_t)


def _make_sc_kernel(N, O, E):
    assert E % CHUNK == 0
    tot_chunks = E // CHUNK
    base_chunks = tot_chunks // NW
    extra = tot_chunks % NW
    row_grps = N // CHUNK
    row_tail = N - row_grps * CHUNK
    assert row_tail % 8 == 0

    mesh = plsc.VectorSubcoreMesh(core_axis_name="c", subcore_axis_name="s")

    assert base_chunks % 2 == 0

    def body(y2_hbm, ridx_hbm, gidx_hbm, z128_hbm, zhist_hbm,
             s_out, cnt_out,
             s_sh, idxg0, idxg1, idxr0, idxr1, scx0, scx1, rows0, rows1,
             hist_v, sem0, sem1, semi0, semi1, semsc0, semsc1):
        cid = lax.axis_index("c")
        sid = lax.axis_index("s")
        wid = sid * NC + cid
        idxg = (idxg0, idxg1)
        idxr = (idxr0, idxr1)
        scx = (scx0, scx1)
        rows = (rows0, rows1)
        sems = (sem0, sem1)
        semi = (semi0, semi1)
        semsc = (semsc0, semsc1)
        rows_v = rows0

        pltpu.sync_copy(z128_hbm, rows_v)
        pltpu.sync_copy(zhist_hbm, hist_v)

        def zero_grp(g, carry):
            o = (sid + g * NS) * CHUNK
            pltpu.sync_copy(rows_v, s_sh.at[pl.ds(o, CHUNK)])
            return carry

        lax.fori_loop(0, row_grps // NS, zero_grp, 0)
        rem = row_grps % NS

        @pl.when(sid < rem)
        def _zero_rem():
            o = ((row_grps // NS) * NS + sid) * CHUNK
            pltpu.sync_copy(rows_v, s_sh.at[pl.ds(o, CHUNK)])

        if row_tail:
            @pl.when(sid == NS - 1)
            def _zero_tail():
                t0 = row_grps * CHUNK
                pltpu.sync_copy(rows_v.at[pl.ds(0, row_tail)],
                                s_sh.at[pl.ds(t0, row_tail)])
        plsc.subcore_barrier()

        def load_idx(j, b):
            base = (wid * base_chunks + j) * CHUNK
            pltpu.sync_copy(gidx_hbm.at[pl.ds(base, CHUNK)], idxg[b])
            pltpu.sync_copy(ridx_hbm.at[pl.ds(base, CHUNK)], idxr[b])

        def prefetch_idx(j, b):
            base = (wid * base_chunks + j) * CHUNK
            pltpu.make_async_copy(gidx_hbm.at[pl.ds(base, CHUNK)],
                                  idxg[b], semi[b]).start()
            pltpu.make_async_copy(ridx_hbm.at[pl.ds(base, CHUNK)],
                                  idxr[b], semi[b]).start()

        def wait_idx(b):
            pltpu.make_async_copy(gidx_hbm.at[pl.ds(0, CHUNK)],
                                  idxg[b], semi[b]).wait()
            pltpu.make_async_copy(ridx_hbm.at[pl.ds(0, CHUNK)],
                                  idxr[b], semi[b]).wait()

        def start_gather(b):
            pltpu.make_async_copy(y2_hbm.at[idxg[b]], rows[b],
                                  sems[b]).start()

        def hist(b):
            for u in range(CHUNK // L):
                iv = idxr[b][pl.ds(u * L, L)]
                scx[b][pl.ds(u * L, L)] = iv
                cnts, last = plsc.scan_count(iv)
                plsc.addupdate_scatter(hist_v, [iv],
                                       cnts.astype(jnp.float32), mask=last)

        def wait_gather(b):
            pltpu.make_async_copy(y2_hbm.at[idxg[b]], rows[b],
                                  sems[b]).wait()

        def start_scatter(b):
            pltpu.async_copy(rows[b], s_sh.at[scx[b]], semsc[b], add=True)

        def wait_scatter(b):
            pltpu.make_async_copy(rows[b], s_sh.at[scx[b]],
                                  semsc[b]).wait()

        load_idx(0, 0)
        start_gather(0)
        prefetch_idx(1, 1)

        def pair(t, first=False):
            hist(0)
            wait_idx(1)
            if not first:
                wait_scatter(1)
            start_gather(1)
            wait_gather(0)
            start_scatter(0)
            prefetch_idx(2 * t + 2, 0)
            hist(1)
            wait_scatter(0)
            wait_idx(0)
            start_gather(0)
            wait_gather(1)
            start_scatter(1)
            prefetch_idx(2 * t + 3, 1)

        pair(0, first=True)

        def pair_loop(t, carry):
            pair(t)
            return carry

        lax.fori_loop(1, base_chunks // 2, pair_loop, 0)
        wait_gather(0)
        wait_idx(1)
        wait_scatter(1)
        if extra:
            @pl.when(wid < extra)
            def _extra():
                j = NW * base_chunks + wid
                base = j * CHUNK
                pltpu.sync_copy(gidx_hbm.at[pl.ds(base, CHUNK)], idxg0)
                pltpu.sync_copy(ridx_hbm.at[pl.ds(base, CHUNK)], idxr0)
                start_gather(0)
                hist(0)
                wait_gather(0)
                start_scatter(0)
                wait_scatter(0)
        plsc.subcore_barrier()

        def pub_grp(g, carry):
            o = (sid + g * NS) * CHUNK
            pltpu.sync_copy(s_sh.at[pl.ds(o, CHUNK)], rows_v)
            pltpu.sync_copy(rows_v, s_out.at[cid, pl.ds(o, CHUNK)])
            return carry

        lax.fori_loop(0, row_grps // NS, pub_grp, 0)

        @pl.when(sid < rem)
        def _pub_rem():
            o = ((row_grps // NS) * NS + sid) * CHUNK
            pltpu.sync_copy(s_sh.at[pl.ds(o, CHUNK)], rows_v)
            pltpu.sync_copy(rows_v, s_out.at[cid, pl.ds(o, CHUNK)])

        if row_tail:
            @pl.when(sid == NS - 1)
            def _pub_tail():
                t0 = row_grps * CHUNK
                pltpu.sync_copy(s_sh.at[pl.ds(t0, row_tail)],
                                rows_v.at[pl.ds(0, row_tail)])
                pltpu.sync_copy(rows_v.at[pl.ds(0, row_tail)],
                                s_out.at[cid, pl.ds(t0, row_tail)])

        pltpu.sync_copy(hist_v, cnt_out.at[cid, sid])

    return pl.kernel(
        body,
        out_type=[
            jax.ShapeDtypeStruct((NC, N, O), jnp.float32),
            jax.ShapeDtypeStruct((NC, NS, N), jnp.float32),
        ],
        mesh=mesh,
        compiler_params=pltpu.CompilerParams(needs_layout_passes=False),
        scratch_types=[
            pltpu.VMEM_SHARED((N, O), jnp.float32),
            pltpu.VMEM((CHUNK,), jnp.int32),
            pltpu.VMEM((CHUNK,), jnp.int32),
            pltpu.VMEM((CHUNK,), jnp.int32),
            pltpu.VMEM((CHUNK,), jnp.int32),
            pltpu.VMEM((CHUNK,), jnp.int32),
            pltpu.VMEM((CHUNK,), jnp.int32),
            pltpu.VMEM((CHUNK, O), jnp.float32),
            pltpu.VMEM((CHUNK, O), jnp.float32),
            pltpu.VMEM((N,), jnp.float32),
            pltpu.SemaphoreType.DMA,
            pltpu.SemaphoreType.DMA,
            pltpu.SemaphoreType.DMA,
            pltpu.SemaphoreType.DMA,
            pltpu.SemaphoreType.DMA,
            pltpu.SemaphoreType.DMA,
        ],
    )


def _combine_body(y1_ref, s_ref, cnt_ref, b_ref, pw_ref, out_ref):
    s = s_ref[0] + s_ref[1]
    c = jnp.sum(cnt_ref[...], axis=0)[:, None]
    y = y1_ref[...] + b_ref[...]
    tot = c * y + s
    out = tot / jnp.maximum(c, 1.0)
    pw = pw_ref[0, 0]
    out = jnp.where(out >= 0, out, pw * out)
    out_ref[...] = out.T


def _stage3(Y1t, S, CNT, b2, pw2):
    N, O = Y1t.shape
    return pl.pallas_call(
        _combine_body,
        out_shape=jax.ShapeDtypeStruct((O, N), jnp.float32),
    )(Y1t, S, CNT, b2, pw2)


def kernel(in_features, reduce_index, gather_index, W, b, prelu_w):
    X = in_features[0]
    C, N = X.shape
    O = W.shape[0]
    E = reduce_index.shape[0]
    ridx = reduce_index.astype(jnp.int32)
    gidx = gather_index.astype(jnp.int32)
    W1 = W[:, :C]
    W2 = W[:, C:]
    Wd_t = (W1 - W2).T
    W2_t = W2.T

    Y1t, Y2t = _stage1(X, Wd_t, W2_t)

    z128 = jnp.zeros((CHUNK, O), jnp.float32)
    zhist = jnp.zeros((N,), jnp.float32)
    S, CNT = _make_sc_kernel(N, O, E)(Y2t, ridx, gidx, z128, zhist)

    out2d = _stage3(Y1t, S, CNT.reshape(NC * NS, N),
                    b.reshape(1, O), prelu_w.reshape(1, 1))
    return out2d[None]
